# Initial kernel scaffold; baseline (speedup 1.0000x reference)
import jax
import jax.numpy as jnp
from jax.experimental import pallas as pl


def kernel(seq, W, a1, b1, a2, b2, bias, edge_index):
    raise NotImplementedError("write your pallas kernel here")


# speedup vs baseline: 22.7270x; 22.7270x over previous
import functools

import jax
import jax.numpy as jnp
from jax import lax
from jax.experimental import pallas as pl
from jax.experimental.pallas import tpu as pltpu
from jax.experimental.pallas import tpu_sc as plsc

_N = 10000
_F = 128
_E = 320000

_NTILES = 32
_NP = 10240
_RPT = _NP // 16
_EPW = _E // _NTILES
_C = 80
_NIT = _EPW // _C


def _mm_body(seq_ref, w_ref, a_ref, sf_ref, fa_ref, mx_ref):
    sf = jnp.dot(seq_ref[...], w_ref[...], preferred_element_type=jnp.float32)
    sf_ref[...] = sf
    fa = jnp.dot(sf, a_ref[...], preferred_element_type=jnp.float32)
    fa_ref[...] = fa
    mx_ref[...] = jnp.max(fa, axis=0, keepdims=True)


def _sc_edge_body(sfh, rowh, colh, f1h, f2h, mh, valsp, denomp,
                  f1b, f2b, mb, rowb, colb, exb, rowsb, vals_s, denom_s, sem):
    c = lax.axis_index("c")
    s = lax.axis_index("s")
    wid = s * 2 + c
    zeros16 = jnp.zeros((16,), jnp.float32)

    def zrow(i, carry):
        for k in range(_F // 16):
            rowsb[i, pl.ds(k * 16, 16)] = zeros16
        return carry
    lax.fori_loop(0, _C, zrow, 0)
    for k in range(_C // 16):
        exb[pl.ds(k * 16, 16)] = zeros16
    for k in range(_RPT // _C):
        pltpu.sync_copy(rowsb, vals_s.at[pl.ds(s * _RPT + k * _C, _C), :])
        pltpu.sync_copy(exb, denom_s.at[pl.ds(s * _RPT + k * _C, _C)])

    pltpu.sync_copy(f1h, f1b)
    pltpu.sync_copy(f2h, f2b)
    pltpu.sync_copy(mh, mb)
    plsc.subcore_barrier()
    mv = mb[...]

    ebase = wid * _EPW

    def step(i, carry):
        base = ebase + i * _C
        pltpu.sync_copy(rowh.at[pl.ds(base, _C)], rowb)
        pltpu.sync_copy(colh.at[pl.ds(base, _C)], colb)
        pltpu.async_copy(sfh.at[colb], rowsb, sem).wait()
        for j in range(_C // 16):
            rv = rowb[pl.ds(j * 16, 16)]
            cv = colb[pl.ds(j * 16, 16)]
            x = plsc.load_gather(f1b, [rv]) + plsc.load_gather(f2b, [cv])
            lr = jnp.maximum(x, 0.2 * x)
            exb[pl.ds(j * 16, 16)] = jnp.exp(lr - mv)

        def scale(i2, carry2):
            ev = plsc.load_gather(exb, [jnp.full((16,), i2, jnp.int32)])
            for k in range(_F // 16):
                rowsb[i2, pl.ds(k * 16, 16)] = rowsb[i2, pl.ds(k * 16, 16)] * ev
            return carry2
        lax.fori_loop(0, _C, scale, 0)

        pltpu.sync_copy(rowsb, vals_s.at[rowb], add=True)
        pltpu.sync_copy(exb, denom_s.at[rowb], add=True)
        return carry
    lax.fori_loop(0, _NIT, step, 0)

    plsc.subcore_barrier()
    pltpu.sync_copy(vals_s.at[pl.ds(s * _RPT, _RPT), :],
                    valsp.at[c, pl.ds(s * _RPT, _RPT), :])
    pltpu.sync_copy(denom_s.at[pl.ds(s * _RPT, _RPT)],
                    denomp.at[c, pl.ds(s * _RPT, _RPT)])


_sc_edge = functools.partial(
    pl.kernel,
    out_type=[jax.ShapeDtypeStruct((2, _NP, _F), jnp.float32),
              jax.ShapeDtypeStruct((2, _NP), jnp.float32)],
    mesh=plsc.VectorSubcoreMesh(core_axis_name="c", subcore_axis_name="s"),
    compiler_params=pltpu.CompilerParams(needs_layout_passes=False),
    scratch_types=[
        pltpu.VMEM((_N,), jnp.float32),
        pltpu.VMEM((_N,), jnp.float32),
        pltpu.VMEM((16,), jnp.float32),
        pltpu.VMEM((_C,), jnp.int32),
        pltpu.VMEM((_C,), jnp.int32),
        pltpu.VMEM((_C,), jnp.float32),
        pltpu.VMEM((_C, _F), jnp.float32),
        pltpu.VMEM_SHARED((_NP, _F), jnp.float32),
        pltpu.VMEM_SHARED((_NP,), jnp.float32),
        pltpu.SemaphoreType.DMA,
    ],
)(_sc_edge_body)


def _fin_body(v_ref, d_ref, b_ref, o_ref):
    v = v_ref[0] + v_ref[1]
    d = (d_ref[0] + d_ref[1] + 1e-16)[:, None]
    o_ref[...] = jnp.maximum(v / d + b_ref[...], 0.0)


def kernel(seq, W, a1, b1, a2, b2, bias, edge_index):
    n, f = seq.shape
    seq = seq.astype(jnp.float32)
    A = jnp.zeros((f, _F), jnp.float32)
    A = A.at[:, 0].set(a1[:, 0]).at[:, 1].set(a2[:, 0])

    sf, fa, mx = pl.pallas_call(
        _mm_body,
        out_shape=[jax.ShapeDtypeStruct((n, _F), jnp.float32),
                   jax.ShapeDtypeStruct((n, _F), jnp.float32),
                   jax.ShapeDtypeStruct((1, _F), jnp.float32)],
    )(seq, W.astype(jnp.float32), A)

    f1 = fa[:, 0] + b1[0]
    f2 = fa[:, 1] + b2[0]
    mval = mx[0, 0] + mx[0, 1] + b1[0] + b2[0]
    m = jnp.maximum(mval, 0.2 * mval)
    marr = jnp.full((16,), m, jnp.float32)

    row = edge_index[0]
    col = edge_index[1]
    valsp, denomp = _sc_edge(sf, row, col, f1, f2, marr)

    blk = 1024
    out = pl.pallas_call(
        _fin_body,
        grid=(_NP // blk,),
        in_specs=[pl.BlockSpec((2, blk, _F), lambda i: (0, i, 0)),
                  pl.BlockSpec((2, blk), lambda i: (0, i)),
                  pl.BlockSpec((1, _F), lambda i: (0, 0))],
        out_specs=pl.BlockSpec((blk, _F), lambda i: (i, 0)),
        out_shape=jax.ShapeDtypeStruct((_NP, _F), jnp.float32),
    )(valsp, denomp, bias.reshape(1, _F))
    return out[:n]
